# baseline (speedup 1.0000x reference)
import functools

import jax
import jax.numpy as jnp
from jax import lax
from jax.experimental import pallas as pl
from jax.experimental.pallas import tpu as pltpu
from jax.experimental.pallas import tpu_sc as plsc

EMB = 64
NLANE = 16
NVEC = EMB // NLANE
NW = 32
CHUNK = 256
GATHER = 128
RU = 8


def _make_sc_kernel(n_rows: int, seq_len: int):
    per_w = n_rows // NW
    n_chunk = per_w // CHUNK
    assert n_rows % NW == 0 and per_w % CHUNK == 0 and n_chunk % 2 == 0
    mesh = plsc.VectorSubcoreMesh(core_axis_name="c", subcore_axis_name="s")

    @functools.partial(
        pl.kernel,
        mesh=mesh,
        compiler_params=pltpu.CompilerParams(needs_layout_passes=False,
                                             use_tc_tiling_on_sc=False),
        out_type=jax.ShapeDtypeStruct((n_rows, EMB), jnp.float32),
        scratch_types=[
            pltpu.VMEM((per_w,), jnp.int32),
            pltpu.VMEM((CHUNK, EMB), jnp.float32),
            pltpu.VMEM((CHUNK, EMB), jnp.float32),
            pltpu.VMEM((CHUNK, EMB), jnp.float32),
            pltpu.VMEM((CHUNK, EMB), jnp.float32),
            pltpu.VMEM((seq_len, EMB), jnp.float32),
            pltpu.VMEM((EMB,), jnp.float32),
            pltpu.VMEM((EMB,), jnp.float32),
            pltpu.SemaphoreType.DMA,
            pltpu.SemaphoreType.DMA,
            pltpu.SemaphoreType.DMA,
            pltpu.SemaphoreType.DMA,
        ],
    )
    def sc_kernel(idx_hbm, tok_hbm, pos_hbm, gam_hbm, bet_hbm, out_hbm,
                  idxall_v, ib0, ib1, ob0, ob1, pos_v, gam_v, bet_v,
                  gsem0, gsem1, osem0, osem1):
        wid = lax.axis_index("s") * 2 + lax.axis_index("c")
        base_w = wid * per_w

        pltpu.sync_copy(idx_hbm.at[pl.ds(base_w, per_w)], idxall_v)
        pltpu.sync_copy(pos_hbm, pos_v)
        pltpu.sync_copy(gam_hbm, gam_v)
        pltpu.sync_copy(bet_hbm, bet_v)
        gv = [gam_v[pl.ds(16 * k, 16)] for k in range(NVEC)]
        bv = [bet_v[pl.ds(16 * k, 16)] for k in range(NVEC)]
        def hsum(v):
            return jnp.sum(v)

        def vrsqrt(v):
            i = lax.bitcast_convert_type(v, jnp.int32)
            i = jnp.int32(0x5F3759DF) - lax.shift_right_arithmetic(i, 1)
            y = lax.bitcast_convert_type(i, jnp.float32)
            vh = 0.5 * v
            for _ in range(2):
                y = y * (1.5 - vh * y * y)
            return y

        ibufs, obufs = (ib0, ib1), (ob0, ob1)
        gsems, osems = (gsem0, gsem1), (osem0, osem1)

        def issue_gather(c, b):
            off = c * CHUNK
            for j in range(CHUNK // GATHER):
                pltpu.async_copy(
                    tok_hbm.at[idxall_v.at[pl.ds(off + j * GATHER, GATHER)]],
                    ibufs[b].at[pl.ds(j * GATHER, GATHER)], gsems[b])

        def wait_gather(b):
            pltpu.make_async_copy(tok_hbm.at[pl.ds(0, CHUNK)],
                                  ibufs[b], gsems[b]).wait()

        def issue_out(c, b):
            pltpu.async_copy(obufs[b],
                             out_hbm.at[pl.ds(base_w + c * CHUNK, CHUNK)],
                             osems[b])

        def wait_out(b):
            pltpu.make_async_copy(obufs[b], out_hbm.at[pl.ds(0, CHUNK)],
                                  osems[b]).wait()

        def compute(c, b):
            base = base_w + c * CHUNK
            ib, ob = ibufs[b], obufs[b]
            p0 = lax.rem(base, seq_len)

            @plsc.parallel_loop(0, CHUNK, step=1, unroll=RU)
            def row_body(r):
                p = p0 + r
                p = jnp.where(p >= seq_len, p - seq_len, p)
                p = jnp.where(p >= seq_len, p - seq_len, p)
                xs = [ib[r, pl.ds(16 * k, 16)] + pos_v[p, pl.ds(16 * k, 16)]
                      for k in range(NVEC)]
                s = (xs[0] + xs[1]) + (xs[2] + xs[3])
                q = (xs[0] * xs[0] + xs[1] * xs[1]) + \
                    (xs[2] * xs[2] + xs[3] * xs[3])
                mean = hsum(s) * (1.0 / EMB)
                var = hsum(q) * (1.0 / EMB) - mean * mean
                rstd = vrsqrt(var + 1e-5)
                m2 = mean * rstd
                for k in range(NVEC):
                    y = xs[k] * rstd - m2
                    ob[r, pl.ds(16 * k, 16)] = y * gv[k] + bv[k]

        issue_gather(0, 0)
        issue_gather(1, 1)
        for b in (0, 1):
            wait_gather(b)
            compute(jnp.int32(b), b)
            issue_out(jnp.int32(b), b)
            issue_gather(jnp.int32(b + 2), b)

        def loop_body(i, carry):
            c0 = 2 * i
            for b in (0, 1):
                c = c0 + b
                wait_gather(b)
                wait_out(b)
                compute(c, b)
                issue_out(c, b)
                issue_gather(jnp.minimum(c + 2, n_chunk - 1), b)
            return carry

        lax.fori_loop(1, n_chunk // 2, loop_body, 0)

        for b in (0, 1):
            wait_gather(b)
            wait_out(b)

    return sc_kernel


def kernel(input_tensor, res_mask, token_table, position_table, gamma, beta):
    b, seq_len = input_tensor.shape
    n_rows = b * seq_len
    idx_flat = input_tensor.reshape(n_rows).astype(jnp.int32)
    pos_used = position_table[:seq_len]
    out = _make_sc_kernel(n_rows, seq_len)(
        idx_flat, token_table, pos_used, gamma, beta)
    return out.reshape(b, seq_len, EMB)


# speedup vs baseline: 1.0129x; 1.0129x over previous
import functools

import jax
import jax.numpy as jnp
from jax import lax
from jax.experimental import pallas as pl
from jax.experimental.pallas import tpu as pltpu
from jax.experimental.pallas import tpu_sc as plsc

EMB = 64
NLANE = 16
NVEC = EMB // NLANE
NW = 32
CHUNK = 128
NB = 3
RU = 4


def _make_sc_kernel(n_rows: int, seq_len: int):
    per_w = n_rows // NW
    n_chunk = per_w // CHUNK
    assert n_rows % NW == 0 and per_w % CHUNK == 0
    assert (n_chunk - 5) % NB == 0 and n_chunk > 5
    mesh = plsc.VectorSubcoreMesh(core_axis_name="c", subcore_axis_name="s")

    @functools.partial(
        pl.kernel,
        mesh=mesh,
        compiler_params=pltpu.CompilerParams(needs_layout_passes=False,
                                             use_tc_tiling_on_sc=False),
        out_type=jax.ShapeDtypeStruct((n_rows, EMB), jnp.float32),
        scratch_types=[
            pltpu.VMEM((per_w,), jnp.int32),
            [pltpu.VMEM((CHUNK, EMB), jnp.float32) for _ in range(NB)],
            [pltpu.VMEM((CHUNK, EMB), jnp.float32) for _ in range(NB)],
            pltpu.VMEM((seq_len, EMB), jnp.float32),
            pltpu.VMEM((EMB,), jnp.float32),
            pltpu.VMEM((EMB,), jnp.float32),
            [pltpu.SemaphoreType.DMA for _ in range(NB)],
            [pltpu.SemaphoreType.DMA for _ in range(NB)],
        ],
    )
    def sc_kernel(idx_hbm, tok_hbm, pos_hbm, gam_hbm, bet_hbm, out_hbm,
                  idxall_v, ibufs, obufs, pos_v, gam_v, bet_v, gsems, osems):
        wid = lax.axis_index("s") * 2 + lax.axis_index("c")
        base_w = wid * per_w

        pltpu.sync_copy(idx_hbm.at[pl.ds(base_w, per_w)], idxall_v)
        pltpu.sync_copy(pos_hbm, pos_v)
        pltpu.sync_copy(gam_hbm, gam_v)
        pltpu.sync_copy(bet_hbm, bet_v)
        gv = [gam_v[pl.ds(16 * k, 16)] for k in range(NVEC)]
        bv = [bet_v[pl.ds(16 * k, 16)] for k in range(NVEC)]

        def vrsqrt(v):
            i = lax.bitcast_convert_type(v, jnp.int32)
            i = jnp.int32(0x5F3759DF) - lax.shift_right_arithmetic(i, 1)
            y = lax.bitcast_convert_type(i, jnp.float32)
            vh = 0.5 * v
            for _ in range(2):
                y = y * (1.5 - vh * y * y)
            return y

        def issue_gather(c, u):
            pltpu.async_copy(
                tok_hbm.at[idxall_v.at[pl.ds(c * CHUNK, CHUNK)]],
                ibufs[u], gsems[u])

        def wait_gather(u):
            pltpu.make_async_copy(tok_hbm.at[pl.ds(0, CHUNK)],
                                  ibufs[u], gsems[u]).wait()

        def issue_out(c, u):
            pltpu.async_copy(obufs[u],
                             out_hbm.at[pl.ds(base_w + c * CHUNK, CHUNK)],
                             osems[u])

        def wait_out(u):
            pltpu.make_async_copy(obufs[u], out_hbm.at[pl.ds(0, CHUNK)],
                                  osems[u]).wait()

        def compute(c, u):
            base = base_w + c * CHUNK
            ib, ob = ibufs[u], obufs[u]
            p0 = lax.rem(base, seq_len)

            @plsc.parallel_loop(0, CHUNK, step=1, unroll=RU)
            def row_body(r):
                p = p0 + r
                p = jnp.where(p >= seq_len, p - seq_len, p)
                xs = [ib[r, pl.ds(16 * k, 16)] + pos_v[p, pl.ds(16 * k, 16)]
                      for k in range(NVEC)]
                s = (xs[0] + xs[1]) + (xs[2] + xs[3])
                q = (xs[0] * xs[0] + xs[1] * xs[1]) + \
                    (xs[2] * xs[2] + xs[3] * xs[3])
                mean = jnp.sum(s) * (1.0 / EMB)
                var = jnp.sum(q) * (1.0 / EMB) - mean * mean
                rstd = vrsqrt(var + 1e-5)
                m2 = mean * rstd
                for k in range(NVEC):
                    y = xs[k] * rstd - m2
                    ob[r, pl.ds(16 * k, 16)] = y * gv[k] + bv[k]

        def run_phase(c, u, do_wait_out):
            wait_gather(u)
            issue_gather(jnp.minimum(c + 2, n_chunk - 1), (u + 2) % NB)
            if do_wait_out:
                wait_out(u)
            compute(c, u)
            issue_out(c, u)

        issue_gather(0, 0)
        issue_gather(1, 1)
        for c in range(5):
            run_phase(jnp.int32(c), c % NB, do_wait_out=(c >= NB))

        def loop_body(i, carry):
            cbase = NB * i + 5
            for t in range(NB):
                run_phase(cbase + t, (5 + t) % NB, do_wait_out=True)
            return carry

        lax.fori_loop(0, (n_chunk - 5) // NB, loop_body, 0)

        for u in (((n_chunk - 2) % NB + 2) % NB, ((n_chunk - 1) % NB + 2) % NB):
            wait_gather(u)
        for c in range(n_chunk - NB, n_chunk):
            wait_out(c % NB)

    return sc_kernel


def kernel(input_tensor, res_mask, token_table, position_table, gamma, beta):
    b, seq_len = input_tensor.shape
    n_rows = b * seq_len
    idx_flat = input_tensor.reshape(n_rows).astype(jnp.int32)
    pos_used = position_table[:seq_len]
    out = _make_sc_kernel(n_rows, seq_len)(
        idx_flat, token_table, pos_used, gamma, beta)
    return out.reshape(b, seq_len, EMB)


# speedup vs baseline: 1.0362x; 1.0230x over previous
import functools

import jax
import jax.numpy as jnp
from jax import lax
from jax.experimental import pallas as pl
from jax.experimental.pallas import tpu as pltpu
from jax.experimental.pallas import tpu_sc as plsc

EMB = 64
NLANE = 16
NVEC = EMB // NLANE
NW = 32
CHUNK = 320
GATHER = 128
RU = 4


def _make_sc_kernel(n_rows: int, seq_len: int):
    per_w = n_rows // NW
    n_chunk = per_w // CHUNK
    assert n_rows % NW == 0 and per_w % CHUNK == 0 and n_chunk % 2 == 0
    mesh = plsc.VectorSubcoreMesh(core_axis_name="c", subcore_axis_name="s")

    @functools.partial(
        pl.kernel,
        mesh=mesh,
        compiler_params=pltpu.CompilerParams(needs_layout_passes=False,
                                             use_tc_tiling_on_sc=False),
        out_type=jax.ShapeDtypeStruct((n_rows, EMB), jnp.float32),
        scratch_types=[
            pltpu.VMEM((per_w,), jnp.int32),
            pltpu.VMEM((CHUNK, EMB), jnp.float32),
            pltpu.VMEM((CHUNK, EMB), jnp.float32),
            pltpu.VMEM((CHUNK, EMB), jnp.float32),
            pltpu.VMEM((CHUNK, EMB), jnp.float32),
            pltpu.VMEM((seq_len, EMB), jnp.float32),
            pltpu.VMEM((EMB,), jnp.float32),
            pltpu.VMEM((EMB,), jnp.float32),
            pltpu.SemaphoreType.DMA,
            pltpu.SemaphoreType.DMA,
            pltpu.SemaphoreType.DMA,
            pltpu.SemaphoreType.DMA,
        ],
    )
    def sc_kernel(idx_hbm, tok_hbm, pos_hbm, gam_hbm, bet_hbm, out_hbm,
                  idxall_v, ib0, ib1, ob0, ob1, pos_v, gam_v, bet_v,
                  gsem0, gsem1, osem0, osem1):
        wid = lax.axis_index("s") * 2 + lax.axis_index("c")
        base_w = wid * per_w

        pltpu.sync_copy(idx_hbm.at[pl.ds(base_w, per_w)], idxall_v)
        pltpu.sync_copy(pos_hbm, pos_v)
        pltpu.sync_copy(gam_hbm, gam_v)
        pltpu.sync_copy(bet_hbm, bet_v)
        gv = [gam_v[pl.ds(16 * k, 16)] for k in range(NVEC)]
        bv = [bet_v[pl.ds(16 * k, 16)] for k in range(NVEC)]
        def hsum(v):
            return jnp.sum(v)

        def vrsqrt(v):
            i = lax.bitcast_convert_type(v, jnp.int32)
            i = jnp.int32(0x5F3759DF) - lax.shift_right_arithmetic(i, 1)
            y = lax.bitcast_convert_type(i, jnp.float32)
            vh = 0.5 * v
            for _ in range(2):
                y = y * (1.5 - vh * y * y)
            return y

        ibufs, obufs = (ib0, ib1), (ob0, ob1)
        gsems, osems = (gsem0, gsem1), (osem0, osem1)

        def issue_gather(c, b):
            off = c * CHUNK
            done = 0
            while done < CHUNK:
                g = min(GATHER, CHUNK - done)
                pltpu.async_copy(
                    tok_hbm.at[idxall_v.at[pl.ds(off + done, g)]],
                    ibufs[b].at[pl.ds(done, g)], gsems[b])
                done += g

        def wait_gather(b):
            pltpu.make_async_copy(tok_hbm.at[pl.ds(0, CHUNK)],
                                  ibufs[b], gsems[b]).wait()

        def issue_out(c, b):
            pltpu.async_copy(obufs[b],
                             out_hbm.at[pl.ds(base_w + c * CHUNK, CHUNK)],
                             osems[b])

        def wait_out(b):
            pltpu.make_async_copy(obufs[b], out_hbm.at[pl.ds(0, CHUNK)],
                                  osems[b]).wait()

        def compute(c, b):
            base = base_w + c * CHUNK
            ib, ob = ibufs[b], obufs[b]
            p0 = lax.rem(base, seq_len)

            @plsc.parallel_loop(0, CHUNK, step=1, unroll=RU)
            def row_body(r):
                p = p0 + r
                p = jnp.where(p >= seq_len, p - seq_len, p)
                p = jnp.where(p >= seq_len, p - seq_len, p)
                xs = [ib[r, pl.ds(16 * k, 16)] + pos_v[p, pl.ds(16 * k, 16)]
                      for k in range(NVEC)]
                s = (xs[0] + xs[1]) + (xs[2] + xs[3])
                q = (xs[0] * xs[0] + xs[1] * xs[1]) + \
                    (xs[2] * xs[2] + xs[3] * xs[3])
                mean = hsum(s) * (1.0 / EMB)
                var = hsum(q) * (1.0 / EMB) - mean * mean
                rstd = vrsqrt(var + 1e-5)
                m2 = mean * rstd
                for k in range(NVEC):
                    y = xs[k] * rstd - m2
                    ob[r, pl.ds(16 * k, 16)] = y * gv[k] + bv[k]

        issue_gather(0, 0)
        issue_gather(1, 1)
        for b in (0, 1):
            wait_gather(b)
            compute(jnp.int32(b), b)
            issue_out(jnp.int32(b), b)
            issue_gather(jnp.int32(b + 2), b)

        def loop_body(i, carry):
            c0 = 2 * i
            for b in (0, 1):
                c = c0 + b
                wait_gather(b)
                wait_out(b)
                compute(c, b)
                issue_out(c, b)
                issue_gather(jnp.minimum(c + 2, n_chunk - 1), b)
            return carry

        lax.fori_loop(1, n_chunk // 2, loop_body, 0)

        for b in (0, 1):
            wait_gather(b)
            wait_out(b)

    return sc_kernel


def kernel(input_tensor, res_mask, token_table, position_table, gamma, beta):
    b, seq_len = input_tensor.shape
    n_rows = b * seq_len
    idx_flat = input_tensor.reshape(n_rows).astype(jnp.int32)
    pos_used = position_table[:seq_len]
    out = _make_sc_kernel(n_rows, seq_len)(
        idx_flat, token_table, pos_used, gamma, beta)
    return out.reshape(b, seq_len, EMB)
